# baseline (speedup 1.0000x reference)
import jax
import jax.numpy as jnp
from jax.experimental import pallas as pl
from jax.experimental.pallas import tpu as pltpu

_B, _MD, _CD, _H, _E = 4096, 138, 11, 512, 8
_ID = _MD + _CD
_BB = 512


def _elu(v):
    return jnp.where(v > 0, v, jnp.exp(jnp.minimum(v, 0.0)) - 1.0)


def _moe_body(x_ref, gW1_ref, gb1_ref, gW2_ref, gb2_ref,
              W0_ref, b0_ref, W1_ref, b1_ref, W2_ref, b2_ref,
              W3_ref, b3_ref, out_ref):
    f32 = jnp.float32
    bf = jnp.bfloat16
    x = x_ref[...]

    h = jnp.dot(x, gW1_ref[...], preferred_element_type=f32) + gb1_ref[...]
    h = _elu(h)
    logits = (jnp.dot(h.astype(bf), gW2_ref[...], preferred_element_type=f32)
              + gb2_ref[...])
    m = jnp.max(logits, axis=-1, keepdims=True)
    p = jnp.exp(logits - m)
    coeffs = p / jnp.sum(p, axis=-1, keepdims=True)
    cb = coeffs.astype(bf)

    def layer0(inp_bf, W_ref, b_ref):
        acc = jnp.dot(cb, b_ref[...].astype(bf), preferred_element_type=f32)
        for e in range(_E):
            me = jnp.dot(inp_bf, W_ref[e], preferred_element_type=f32)
            acc = acc + coeffs[:, e:e + 1] * me
        return _elu(acc)

    def layer(inp_bf, W_ref, b_ref, act):
        t = jnp.concatenate([inp_bf * cb[:, e:e + 1] for e in range(_E)],
                            axis=1)
        acc = jnp.dot(t, W_ref[...], preferred_element_type=f32)
        acc = acc + jnp.dot(cb, b_ref[...].astype(bf),
                            preferred_element_type=f32)
        if act:
            acc = _elu(acc)
        return acc

    y = layer0(x, W0_ref, b0_ref)
    y = layer(y.astype(bf), W1_ref, b1_ref, True)
    y = layer(y.astype(bf), W2_ref, b2_ref, True)
    y = layer(y.astype(bf), W3_ref, b3_ref, False)
    out_ref[...] = y


def kernel(motion, command, gW1, gb1, gW2, gb2, W0, b0, W1, b1, W2, b2, W3, b3):
    bf = jnp.bfloat16
    x = jnp.concatenate([motion, command], axis=-1).astype(bf)
    gW1b = gW1.astype(bf)
    gW2b = gW2.astype(bf)
    Wt0 = W0.transpose(0, 2, 1).astype(bf)
    Wt1 = W1.transpose(0, 2, 1).astype(bf).reshape(_E * _H, _H)
    Wt2 = W2.transpose(0, 2, 1).astype(bf).reshape(_E * _H, _H)
    Wt3 = W3.transpose(0, 2, 1).astype(bf).reshape(_E * _H, _MD)

    grid = (_B // _BB,)
    const2 = lambda i: (0, 0)
    const3 = lambda i: (0, 0, 0)
    in_specs = [
        pl.BlockSpec((_BB, _ID), lambda i: (i, 0)),
        pl.BlockSpec((_ID, _H), const2),
        pl.BlockSpec((1, _H), const2),
        pl.BlockSpec((_H, _E), const2),
        pl.BlockSpec((1, _E), const2),
        pl.BlockSpec((_E, _ID, _H), const3),
        pl.BlockSpec((_E, _H), const2),
        pl.BlockSpec((_E * _H, _H), const2),
        pl.BlockSpec((_E, _H), const2),
        pl.BlockSpec((_E * _H, _H), const2),
        pl.BlockSpec((_E, _H), const2),
        pl.BlockSpec((_E * _H, _MD), const2),
        pl.BlockSpec((_E, _MD), const2),
    ]
    out = pl.pallas_call(
        _moe_body,
        grid=grid,
        in_specs=in_specs,
        out_specs=pl.BlockSpec((_BB, _MD), lambda i: (i, 0)),
        out_shape=jax.ShapeDtypeStruct((_B, _MD), jnp.float32),
        compiler_params=pltpu.CompilerParams(
            dimension_semantics=("parallel",),
        ),
    )(x, gW1b, gb1.reshape(1, _H), gW2b, gb2.reshape(1, _E),
      Wt0, b0, Wt1, b1, Wt2, b2, Wt3, b3)
    return out


# speedup vs baseline: 1.1876x; 1.1876x over previous
import jax
import jax.numpy as jnp
from jax.experimental import pallas as pl
from jax.experimental.pallas import tpu as pltpu

_B, _MD, _CD, _H, _E = 4096, 138, 11, 512, 8
_ID = _MD + _CD
_BB = 512


def _elu(v):
    return jnp.where(v > 0, v, jnp.exp(jnp.minimum(v, 0.0)) - 1.0)


def _moe_body(x_ref, g1_ref, gb1_ref, g2_ref, gb2_ref,
              W0_ref, b0_ref, W1_ref, b1_ref, W2_ref, b2_ref,
              W3_ref, b3_ref, out_ref):
    f32 = jnp.float32
    bf = jnp.bfloat16
    xt = x_ref[...]

    h = jnp.dot(g1_ref[...], xt, preferred_element_type=f32) + gb1_ref[...]
    h = _elu(h)
    logits = (jnp.dot(g2_ref[...], h.astype(bf), preferred_element_type=f32)
              + gb2_ref[...])
    m = jnp.max(logits, axis=0, keepdims=True)
    p = jnp.exp(logits - m)
    coeffs = p / jnp.sum(p, axis=0, keepdims=True)
    cb = coeffs.astype(bf)

    def layer(inp_bf, W_ref, bt_ref, act):
        acc = jnp.dot(bt_ref[...], cb, preferred_element_type=f32)
        for e in range(_E):
            me = jnp.dot(W_ref[e], inp_bf, preferred_element_type=f32)
            acc = acc + coeffs[e:e + 1, :] * me
        if act:
            acc = _elu(acc)
        return acc

    y = layer(xt, W0_ref, b0_ref, True)
    y = layer(y.astype(bf), W1_ref, b1_ref, True)
    y = layer(y.astype(bf), W2_ref, b2_ref, True)
    y = layer(y.astype(bf), W3_ref, b3_ref, False)
    out_ref[...] = y


def kernel(motion, command, gW1, gb1, gW2, gb2, W0, b0, W1, b1, W2, b2, W3, b3):
    bf = jnp.bfloat16
    xt = jnp.concatenate([motion, command], axis=-1).T.astype(bf)
    g1 = gW1.T.astype(bf)
    g2 = gW2.T.astype(bf)
    W0b, W1b, W2b = W0.astype(bf), W1.astype(bf), W2.astype(bf)
    W3b = W3.astype(bf)
    b0t, b1t, b2t = b0.T.astype(bf), b1.T.astype(bf), b2.T.astype(bf)
    b3t = b3.T.astype(bf)

    grid = (_B // _BB,)
    const2 = lambda i: (0, 0)
    const3 = lambda i: (0, 0, 0)
    in_specs = [
        pl.BlockSpec((_ID, _BB), lambda i: (0, i)),
        pl.BlockSpec((_H, _ID), const2),
        pl.BlockSpec((_H, 1), const2),
        pl.BlockSpec((_E, _H), const2),
        pl.BlockSpec((_E, 1), const2),
        pl.BlockSpec((_E, _H, _ID), const3),
        pl.BlockSpec((_H, _E), const2),
        pl.BlockSpec((_E, _H, _H), const3),
        pl.BlockSpec((_H, _E), const2),
        pl.BlockSpec((_E, _H, _H), const3),
        pl.BlockSpec((_H, _E), const2),
        pl.BlockSpec((_E, _MD, _H), const3),
        pl.BlockSpec((_MD, _E), const2),
    ]
    out = pl.pallas_call(
        _moe_body,
        grid=grid,
        in_specs=in_specs,
        out_specs=pl.BlockSpec((_MD, _BB), lambda i: (0, i)),
        out_shape=jax.ShapeDtypeStruct((_MD, _B), jnp.float32),
        compiler_params=pltpu.CompilerParams(
            dimension_semantics=("arbitrary",),
        ),
    )(xt, g1, gb1.reshape(_H, 1), g2, gb2.reshape(_E, 1),
      W0b, b0t, W1b, b1t, W2b, b2t, W3b, b3t)
    return out.T


# speedup vs baseline: 1.2874x; 1.0840x over previous
import jax
import jax.numpy as jnp
from jax.experimental import pallas as pl
from jax.experimental.pallas import tpu as pltpu

_B, _MD, _CD, _H, _E = 4096, 138, 11, 512, 8
_ID = _MD + _CD
_BB = 512
_EH = _E * _H


def _elu(v):
    return jnp.where(v > 0, v, jnp.exp(jnp.minimum(v, 0.0)) - 1.0)


def _moe_body(motion_ref, command_ref, gW1_ref, gb1_ref, gW2_ref, gb2_ref,
              W0_ref, b0_ref, W1_ref, b1_ref, W2_ref, b2_ref,
              W3_ref, b3_ref, out_ref,
              g1s, g1b, g2s, g2b, W0s, b0s, Wc1, b1s, Wc2, b2s, Wc3, b3s, rs):
    f32 = jnp.float32
    bf = jnp.bfloat16

    @pl.when(pl.program_id(0) == 0)
    def _prep():
        g1s[...] = gW1_ref[...].T.astype(bf)
        g1b[...] = gb1_ref[...].T
        g2s[...] = gW2_ref[...].T.astype(bf)
        g2b[...] = gb2_ref[...].T
        W0s[...] = W0_ref[...].astype(bf)
        b0s[...] = b0_ref[...].T.astype(bf)
        b1s[...] = b1_ref[...].T.astype(bf)
        b2s[...] = b2_ref[...].T.astype(bf)
        b3s[...] = b3_ref[...].T.astype(bf)
        for e in range(_E):
            Wc1[:, e * _H:(e + 1) * _H] = W1_ref[e].astype(bf)
            Wc2[:, e * _H:(e + 1) * _H] = W2_ref[e].astype(bf)
            Wc3[:, e * _H:(e + 1) * _H] = W3_ref[e].astype(bf)

    xt = jnp.concatenate([motion_ref[...].T, command_ref[...].T],
                         axis=0).astype(bf)

    h = jnp.dot(g1s[...], xt, preferred_element_type=f32) + g1b[...]
    h = _elu(h)
    logits = (jnp.dot(g2s[...], h.astype(bf), preferred_element_type=f32)
              + g2b[...])
    m = jnp.max(logits, axis=0, keepdims=True)
    p = jnp.exp(logits - m)
    coeffs = p / jnp.sum(p, axis=0, keepdims=True)
    cb = coeffs.astype(bf)

    acc = jnp.dot(b0s[...], cb, preferred_element_type=f32)
    for e in range(_E):
        me = jnp.dot(W0s[e], xt, preferred_element_type=f32)
        acc = acc + coeffs[e:e + 1, :] * me
    y = _elu(acc)

    def layer(inp_f32, Wc, bs, act):
        inp_bf = inp_f32.astype(bf)
        for e in range(_E):
            rs[e * _H:(e + 1) * _H, :] = inp_bf * cb[e:e + 1, :]
        acc = jnp.dot(Wc[...], rs[...], preferred_element_type=f32)
        acc = acc + jnp.dot(bs[...], cb, preferred_element_type=f32)
        return _elu(acc) if act else acc

    y = layer(y, Wc1, b1s, True)
    y = layer(y, Wc2, b2s, True)
    y = layer(y, Wc3, b3s, False)
    out_ref[...] = y.T


def kernel(motion, command, gW1, gb1, gW2, gb2, W0, b0, W1, b1, W2, b2, W3, b3):
    grid = (_B // _BB,)
    const2 = lambda i: (0, 0)
    const3 = lambda i: (0, 0, 0)
    bf = jnp.bfloat16
    f32 = jnp.float32
    in_specs = [
        pl.BlockSpec((_BB, _MD), lambda i: (i, 0)),
        pl.BlockSpec((_BB, _CD), lambda i: (i, 0)),
        pl.BlockSpec((_ID, _H), const2),
        pl.BlockSpec((1, _H), const2),
        pl.BlockSpec((_H, _E), const2),
        pl.BlockSpec((1, _E), const2),
        pl.BlockSpec((_E, _H, _ID), const3),
        pl.BlockSpec((_E, _H), const2),
        pl.BlockSpec((_E, _H, _H), const3),
        pl.BlockSpec((_E, _H), const2),
        pl.BlockSpec((_E, _H, _H), const3),
        pl.BlockSpec((_E, _H), const2),
        pl.BlockSpec((_E, _MD, _H), const3),
        pl.BlockSpec((_E, _MD), const2),
    ]
    scratch_shapes = [
        pltpu.VMEM((_H, _ID), bf),
        pltpu.VMEM((_H, 1), f32),
        pltpu.VMEM((_E, _H), bf),
        pltpu.VMEM((_E, 1), f32),
        pltpu.VMEM((_E, _H, _ID), bf),
        pltpu.VMEM((_H, _E), bf),
        pltpu.VMEM((_H, _EH), bf),
        pltpu.VMEM((_H, _E), bf),
        pltpu.VMEM((_H, _EH), bf),
        pltpu.VMEM((_H, _E), bf),
        pltpu.VMEM((_MD, _EH), bf),
        pltpu.VMEM((_MD, _E), bf),
        pltpu.VMEM((_EH, _BB), bf),
    ]
    out = pl.pallas_call(
        _moe_body,
        grid=grid,
        in_specs=in_specs,
        out_specs=pl.BlockSpec((_BB, _MD), lambda i: (i, 0)),
        out_shape=jax.ShapeDtypeStruct((_B, _MD), jnp.float32),
        scratch_shapes=scratch_shapes,
        compiler_params=pltpu.CompilerParams(
            dimension_semantics=("arbitrary",),
        ),
    )(motion, command, gW1, gb1.reshape(1, _H), gW2, gb2.reshape(1, _E),
      W0, b0, W1, b1, W2, b2, W3, b3)
    return out
